# baseline (speedup 1.0000x reference)
import math

import jax
import jax.numpy as jnp
import numpy as np
from jax.experimental import pallas as pl
from jax.experimental.pallas import tpu as pltpu

_B = 4
_L = 8192
_DM = 1024
_H = 16
_DK = _DM // _H
_HP = _H // 2
_DP = 2 * _DK
_FACTOR = 5
_TILE = 1024
_PREC = jax.lax.Precision.DEFAULT


def _dot(a, b, dims, prec=_PREC):
    return jax.lax.dot_general(
        a, b, (dims, ((), ())), precision=prec,
        preferred_element_type=jnp.float32)


_MPREC = jax.lax.Precision.DEFAULT


def _proj_m_kernel(idx_ref, q_ref, k_ref, v_ref, khbm_ref,
                   wq_ref, wk_ref, wv_ref, bq_ref, bk_ref, bv_ref,
                   kout_ref, vout_ref, m_ref, ks_scr, sem):
    b = pl.program_id(0)
    t = pl.program_id(1)
    u = ks_scr.shape[0]
    tile = q_ref.shape[1]

    @pl.when(t == 0)
    def _():
        def start(i, c):
            idx = idx_ref[i]
            pltpu.make_async_copy(khbm_ref.at[b, pl.ds(idx, 1), :],
                                  ks_scr.at[pl.ds(i, 1), :], sem).start()
            return c
        jax.lax.fori_loop(0, u, start, 0)

        def wait(i, c):
            pltpu.make_async_copy(khbm_ref.at[b, pl.ds(0, 1), :],
                                  ks_scr.at[pl.ds(i, 1), :], sem).wait()
            return c
        jax.lax.fori_loop(0, u, wait, 0)
        ks_scr[...] = _dot(ks_scr[...], wk_ref[...], ((1,), (1,)), _MPREC) + bk_ref[...]

    kt = (_dot(k_ref[0], wk_ref[...], ((1,), (1,))) + bk_ref[...]
          ).astype(jnp.bfloat16)
    vt = (_dot(v_ref[0], wv_ref[...], ((1,), (1,))) + bv_ref[...]
          ).astype(jnp.bfloat16)
    for g in range(_HP):
        kout_ref[0, g] = kt[:, g * _DP:(g + 1) * _DP]
        vout_ref[0, g] = vt[:, g * _DP:(g + 1) * _DP]

    qt = _dot(q_ref[0], wq_ref[...], ((1,), (1,)), _MPREC) + bq_ref[...]
    ks = ks_scr[...]
    u2 = 2 * u
    colu = jax.lax.broadcasted_iota(jnp.int32, (u, _DP), 1)
    rows = jax.lax.broadcasted_iota(jnp.int32, (u2, tile), 0)
    lo = rows < u
    for g in range(_HP):
        ks_pair = ks[:, g * _DP:(g + 1) * _DP]
        top = jnp.where(colu < _DK, ks_pair, 0.0)
        bot = jnp.where(colu >= _DK, ks_pair, 0.0)
        ks_bd = jnp.concatenate([top, bot], axis=0)
        st = _dot(ks_bd, qt[:, g * _DP:(g + 1) * _DP], ((1,), (1,)), _MPREC)
        m_ref[0, 2 * g, :] = (jnp.max(jnp.where(lo, st, -jnp.inf), axis=0)
                              - jnp.sum(jnp.where(lo, st, 0.0), axis=0) / u)
        m_ref[0, 2 * g + 1, :] = (jnp.max(jnp.where(lo, -jnp.inf, st), axis=0)
                                  - jnp.sum(jnp.where(lo, 0.0, st), axis=0) / u)


def _proj_m(sample_idx, queries, keys, values, wq, wk, wv, bq, bk, bv, u):
    nt = _L // _TILE
    grid_spec = pltpu.PrefetchScalarGridSpec(
        num_scalar_prefetch=1,
        grid=(_B, nt),
        in_specs=[
            pl.BlockSpec((1, _TILE, _DM), lambda b, t, s: (b, t, 0)),
            pl.BlockSpec((1, _TILE, _DM), lambda b, t, s: (b, t, 0)),
            pl.BlockSpec((1, _TILE, _DM), lambda b, t, s: (b, t, 0)),
            pl.BlockSpec(memory_space=pl.ANY),
            pl.BlockSpec((_DM, _DM), lambda b, t, s: (0, 0)),
            pl.BlockSpec((_DM, _DM), lambda b, t, s: (0, 0)),
            pl.BlockSpec((_DM, _DM), lambda b, t, s: (0, 0)),
            pl.BlockSpec((1, _DM), lambda b, t, s: (0, 0)),
            pl.BlockSpec((1, _DM), lambda b, t, s: (0, 0)),
            pl.BlockSpec((1, _DM), lambda b, t, s: (0, 0)),
        ],
        out_specs=[
            pl.BlockSpec((1, _HP, _TILE, _DP), lambda b, t, s: (b, 0, t, 0)),
            pl.BlockSpec((1, _HP, _TILE, _DP), lambda b, t, s: (b, 0, t, 0)),
            pl.BlockSpec((1, _H, _TILE), lambda b, t, s: (b, 0, t)),
        ],
        scratch_shapes=[
            pltpu.VMEM((u, _DM), jnp.float32),
            pltpu.SemaphoreType.DMA,
        ],
    )
    return pl.pallas_call(
        _proj_m_kernel,
        grid_spec=grid_spec,
        out_shape=[
            jax.ShapeDtypeStruct((_B, _HP, _L, _DP), jnp.bfloat16),
            jax.ShapeDtypeStruct((_B, _HP, _L, _DP), jnp.bfloat16),
            jax.ShapeDtypeStruct((_B, _H, _L), jnp.float32),
        ],
        compiler_params=pltpu.CompilerParams(
            dimension_semantics=("arbitrary", "arbitrary")),
    )(sample_idx, queries, keys, values, keys,
      wq, wk, wv, bq.reshape(1, _DM), bk.reshape(1, _DM), bv.reshape(1, _DM))


def _topk_kernel(m_ref, idx_ref, idxf_ref, m_scr, nsel):
    bh = m_ref.shape[0]
    m_scr[...] = m_ref[...]
    iota = jax.lax.broadcasted_iota(jnp.int32, (bh, _L), 1)
    col = jax.lax.broadcasted_iota(jnp.int32, (bh, nsel), 1)

    def body(i, c):
        m = m_scr[...]
        mx = jnp.max(m, axis=1, keepdims=True)
        idx = jnp.min(jnp.where(m == mx, iota, _L), axis=1)
        idx_ref[...] = jnp.where(col == i, idx[:, None], idx_ref[...])
        idxf_ref[...] = jnp.where(col == i, idx.astype(jnp.float32)[:, None],
                                  idxf_ref[...])
        m_scr[...] = jnp.where(iota == idx[:, None], -jnp.inf, m)
        return c
    jax.lax.fori_loop(0, nsel, body, 0)


def _topk(m2, nsel):
    bh = m2.shape[0]
    return pl.pallas_call(
        lambda m_ref, i_ref, f_ref, m_scr: _topk_kernel(
            m_ref, i_ref, f_ref, m_scr, nsel),
        grid=(1,),
        in_specs=[pl.BlockSpec((bh, _L), lambda i: (0, 0))],
        out_specs=[pl.BlockSpec((bh, nsel), lambda i: (0, 0)),
                   pl.BlockSpec((bh, nsel), lambda i: (0, 0))],
        out_shape=[jax.ShapeDtypeStruct((bh, nsel), jnp.int32),
                   jax.ShapeDtypeStruct((bh, nsel), jnp.float32)],
        scratch_shapes=[pltpu.VMEM((bh, _L), jnp.float32)],
    )(m2)


def _attn_kernel(idx_ref, qhbm_ref, k_ref, v_ref, wq_ref, bq_ref,
                 attn_ref, upd_ref, vmean_ref, qg_scr, sems, nsel, scale):
    b = pl.program_id(0)
    g = pl.program_id(1)
    n2 = 2 * nsel
    s = b * _HP + g
    nsteps = _B * _HP

    def issue(sidx, buf):
        bb = sidx // _HP
        gg = sidx - bb * _HP

        def start(i, c):
            half = i // nsel
            idx = idx_ref[bb, 2 * gg + half, i - half * nsel]
            pltpu.make_async_copy(qhbm_ref.at[bb, pl.ds(idx, 1), :],
                                  qg_scr.at[buf, pl.ds(i, 1), :],
                                  sems.at[buf]).start()
            return c
        jax.lax.fori_loop(0, n2, start, 0)

    @pl.when(s == 0)
    def _():
        issue(0, 0)

    @pl.when(s + 1 < nsteps)
    def _():
        issue(s + 1, (s + 1) % 2)

    buf = s % 2

    def wait(i, c):
        pltpu.make_async_copy(qhbm_ref.at[0, pl.ds(0, 1), :],
                              qg_scr.at[buf, pl.ds(i, 1), :],
                              sems.at[buf]).wait()
        return c
    jax.lax.fori_loop(0, n2, wait, 0)

    rows = jax.lax.broadcasted_iota(jnp.int32, (n2, _DP), 0)
    cols = jax.lax.broadcasted_iota(jnp.int32, (n2, _DP), 1)
    mask = (rows < nsel) == (cols < _DK)

    qs = _dot(qg_scr[buf], wq_ref[...], ((1,), (1,))) + bq_ref[0]
    qs_bd = jnp.where(mask, qs, 0.0).astype(jnp.bfloat16)
    scores = _dot(qs_bd, k_ref[0, 0], ((1,), (1,))) * scale
    mx = jnp.max(scores, axis=1, keepdims=True)
    p = jnp.exp(scores - mx)
    a = p / jnp.sum(p, axis=1, keepdims=True)
    attn_ref[0, 0] = a[:nsel]
    attn_ref[0, 1] = a[nsel:]
    upd_ref[0, 0] = _dot(a.astype(jnp.bfloat16), v_ref[0, 0],
                         ((1,), (0,)))
    vmean_ref[0, 0, 0] = jnp.mean(v_ref[0, 0].astype(jnp.float32), axis=0)


def _attn(m_top, queries, k_pair, v_pair, wq, bq, nsel, scale):
    n2 = 2 * nsel
    grid_spec = pltpu.PrefetchScalarGridSpec(
        num_scalar_prefetch=1,
        grid=(_B, _HP),
        in_specs=[
            pl.BlockSpec(memory_space=pl.ANY),
            pl.BlockSpec((1, 1, _L, _DP), lambda b, g, s: (b, g, 0, 0)),
            pl.BlockSpec((1, 1, _L, _DP), lambda b, g, s: (b, g, 0, 0)),
            pl.BlockSpec((_DP, _DM), lambda b, g, s: (g, 0)),
            pl.BlockSpec((1, 1, _DP), lambda b, g, s: (g, 0, 0)),
        ],
        out_specs=[
            pl.BlockSpec((1, 2, nsel, _L), lambda b, g, s: (b, g, 0, 0)),
            pl.BlockSpec((1, 1, n2, _DP), lambda b, g, s: (b, g, 0, 0)),
            pl.BlockSpec((1, 1, 1, _DP), lambda b, g, s: (b, g, 0, 0)),
        ],
        scratch_shapes=[
            pltpu.VMEM((2, n2, _DM), jnp.float32),
            pltpu.SemaphoreType.DMA((2,)),
        ],
    )
    kern = lambda *a: _attn_kernel(*a, nsel=nsel, scale=scale)
    return pl.pallas_call(
        kern,
        grid_spec=grid_spec,
        out_shape=[
            jax.ShapeDtypeStruct((_B, _H, nsel, _L), jnp.float32),
            jax.ShapeDtypeStruct((_B, _HP, n2, _DP), jnp.float32),
            jax.ShapeDtypeStruct((_B, _HP, 1, _DP), jnp.float32),
        ],
        compiler_params=pltpu.CompilerParams(
            dimension_semantics=("arbitrary", "arbitrary")),
    )(m_top, queries, k_pair, v_pair, wq, bq.reshape(_HP, 1, _DP))


def _out_kernel(upd_ref, vm_ref, vmf_ref, wo_ref, bo_ref, idxf_ref,
                out_ref, delta_scr, base_scr, nsel, lh):
    t = pl.program_id(1)
    n2 = 2 * nsel

    @pl.when(t == 0)
    def _():
        base_scr[...] = _dot(vmf_ref[0], wo_ref[...], ((1,), (1,))) + bo_ref[...]
        rows = jax.lax.broadcasted_iota(jnp.int32, (n2, _DP), 0)
        cols = jax.lax.broadcasted_iota(jnp.int32, (n2, _DP), 1)
        mask = (rows < nsel) == (cols < _DK)
        for g in range(_HP):
            du = jnp.where(mask, upd_ref[0, g] - vm_ref[0, g], 0.0)
            wo_g = wo_ref[:, g * _DP:(g + 1) * _DP]
            delta_scr[g * n2:(g + 1) * n2, :] = _dot(du, wo_g, ((1,), (1,)))

    nrow = delta_scr.shape[0]
    lo = t * lh
    rowv = (jax.lax.broadcasted_iota(jnp.int32, (lh, nrow), 0)
            + lo).astype(jnp.float32)
    sel = jnp.where(idxf_ref[0, 0][None, :] == rowv, 1.0, 0.0)
    st = _dot(sel, delta_scr[...], ((1,), (0,)))
    out_ref[0] = st + base_scr[...]


def _out(idxf, upd_pair, vmean_pair, wo, bo, nsel):
    nt = 8
    lh = _L // nt
    n2 = 2 * nsel
    kern = lambda *a: _out_kernel(*a, nsel=nsel, lh=lh)
    return pl.pallas_call(
        kern,
        grid=(_B, nt),
        in_specs=[
            pl.BlockSpec((1, _HP, n2, _DP), lambda b, t: (b, 0, 0, 0)),
            pl.BlockSpec((1, _HP, 1, _DP), lambda b, t: (b, 0, 0, 0)),
            pl.BlockSpec((1, 1, _DM), lambda b, t: (b, 0, 0)),
            pl.BlockSpec((_DM, _DM), lambda b, t: (0, 0)),
            pl.BlockSpec((1, _DM), lambda b, t: (0, 0)),
            pl.BlockSpec((1, 1, _H * nsel), lambda b, t: (b, 0, 0)),
        ],
        out_specs=pl.BlockSpec((1, lh, _DM), lambda b, t: (b, t, 0)),
        out_shape=jax.ShapeDtypeStruct((_B, _L, _DM), jnp.float32),
        scratch_shapes=[
            pltpu.VMEM((_H * nsel, _DM), jnp.float32),
            pltpu.VMEM((1, _DM), jnp.float32),
        ],
        compiler_params=pltpu.CompilerParams(
            dimension_semantics=("arbitrary", "arbitrary")),
    )(upd_pair, vmean_pair, vmean_pair.reshape(_B, 1, _DM),
      wo, bo.reshape(1, _DM), idxf)


def kernel(queries, keys, values, Wq, bq, Wk, bk, Wv, bv, Wo, bo):
    bsz, l_q, _ = queries.shape
    _, l_k, _ = keys.shape
    u = min(_FACTOR * int(np.ceil(np.log(l_k + 1))), l_k)
    nsel = min(_FACTOR * int(np.ceil(np.log(l_q + 1))), l_q)
    scale = 1.0 / math.sqrt(_DK)
    sample_idx = jax.random.randint(jax.random.key(42), (u,), 0, l_k)

    k_pair, v_pair, m = _proj_m(sample_idx, queries, keys, values,
                                Wq, Wk, Wv, bq, bk, bv, u)
    m_top2, m_topf2 = _topk(m.reshape(_B * _H, _L), nsel)
    m_top = m_top2.reshape(_B, _H, nsel)
    idxf = m_topf2.reshape(_B, 1, _H * nsel)
    attn, upd_pair, vmean_pair = _attn(m_top, queries, k_pair, v_pair,
                                       Wq, bq, nsel, scale)
    output = _out(idxf, upd_pair, vmean_pair, Wo, bo, nsel)
    import os as _os
    _stage = 4
    if _stage == 1:
        return m
    if _stage == 2:
        return m_top2
    if _stage == 3:
        return (attn, upd_pair)
    return (output, attn)


# speedup vs baseline: 1.1163x; 1.1163x over previous
import math

import jax
import jax.numpy as jnp
import numpy as np
from jax.experimental import pallas as pl
from jax.experimental.pallas import tpu as pltpu

_B = 4
_L = 8192
_DM = 1024
_H = 16
_DK = _DM // _H
_HP = _H // 2
_DP = 2 * _DK
_FACTOR = 5
_TILE = 512
_PREC = jax.lax.Precision.DEFAULT


def _dot(a, b, dims, prec=_PREC):
    return jax.lax.dot_general(
        a, b, (dims, ((), ())), precision=prec,
        preferred_element_type=jnp.float32)


_MPREC = jax.lax.Precision.DEFAULT


def _proj_m_kernel(idx_ref, q_ref, k_ref, v_ref, khbm_ref,
                   wq_ref, wk_ref, wv_ref, bq_ref, bk_ref, bv_ref,
                   kout_ref, vout_ref, qout_ref, m_ref, ks_scr, sem):
    b = pl.program_id(0)
    t = pl.program_id(1)
    u = ks_scr.shape[0]
    tile = q_ref.shape[1]

    @pl.when(t == 0)
    def _():
        def start(i, c):
            idx = idx_ref[i]
            pltpu.make_async_copy(khbm_ref.at[b, pl.ds(idx, 1), :],
                                  ks_scr.at[pl.ds(i, 1), :], sem).start()
            return c
        jax.lax.fori_loop(0, u, start, 0)

        def wait(i, c):
            pltpu.make_async_copy(khbm_ref.at[b, pl.ds(0, 1), :],
                                  ks_scr.at[pl.ds(i, 1), :], sem).wait()
            return c
        jax.lax.fori_loop(0, u, wait, 0)
        ks_scr[...] = _dot(ks_scr[...], wk_ref[...], ((1,), (1,)), _MPREC) + bk_ref[...]

    kt = (_dot(k_ref[0], wk_ref[...], ((1,), (1,))) + bk_ref[...]
          ).astype(jnp.bfloat16)
    vt = (_dot(v_ref[0], wv_ref[...], ((1,), (1,))) + bv_ref[...]
          ).astype(jnp.bfloat16)
    for g in range(_HP):
        kout_ref[0, g] = kt[:, g * _DP:(g + 1) * _DP]
        vout_ref[0, g] = vt[:, g * _DP:(g + 1) * _DP]

    qt = _dot(q_ref[0], wq_ref[...], ((1,), (1,)), _MPREC) + bq_ref[...]
    qt_bf = qt.astype(jnp.bfloat16)
    for g in range(_HP):
        qout_ref[0, g] = qt_bf[:, g * _DP:(g + 1) * _DP]
    ks = ks_scr[...]
    u2 = 2 * u
    colu = jax.lax.broadcasted_iota(jnp.int32, (u, _DP), 1)
    rows = jax.lax.broadcasted_iota(jnp.int32, (u2, tile), 0)
    lo = rows < u
    for g in range(_HP):
        ks_pair = ks[:, g * _DP:(g + 1) * _DP]
        top = jnp.where(colu < _DK, ks_pair, 0.0)
        bot = jnp.where(colu >= _DK, ks_pair, 0.0)
        ks_bd = jnp.concatenate([top, bot], axis=0)
        st = _dot(ks_bd, qt[:, g * _DP:(g + 1) * _DP], ((1,), (1,)), _MPREC)
        m_ref[0, 2 * g, :] = (jnp.max(jnp.where(lo, st, -jnp.inf), axis=0)
                              - jnp.sum(jnp.where(lo, st, 0.0), axis=0) / u)
        m_ref[0, 2 * g + 1, :] = (jnp.max(jnp.where(lo, -jnp.inf, st), axis=0)
                                  - jnp.sum(jnp.where(lo, 0.0, st), axis=0) / u)


def _proj_m(sample_idx, queries, keys, values, wq, wk, wv, bq, bk, bv, u):
    nt = _L // _TILE
    grid_spec = pltpu.PrefetchScalarGridSpec(
        num_scalar_prefetch=1,
        grid=(_B, nt),
        in_specs=[
            pl.BlockSpec((1, _TILE, _DM), lambda b, t, s: (b, t, 0)),
            pl.BlockSpec((1, _TILE, _DM), lambda b, t, s: (b, t, 0)),
            pl.BlockSpec((1, _TILE, _DM), lambda b, t, s: (b, t, 0)),
            pl.BlockSpec(memory_space=pl.ANY),
            pl.BlockSpec((_DM, _DM), lambda b, t, s: (0, 0)),
            pl.BlockSpec((_DM, _DM), lambda b, t, s: (0, 0)),
            pl.BlockSpec((_DM, _DM), lambda b, t, s: (0, 0)),
            pl.BlockSpec((1, _DM), lambda b, t, s: (0, 0)),
            pl.BlockSpec((1, _DM), lambda b, t, s: (0, 0)),
            pl.BlockSpec((1, _DM), lambda b, t, s: (0, 0)),
        ],
        out_specs=[
            pl.BlockSpec((1, _HP, _TILE, _DP), lambda b, t, s: (b, 0, t, 0)),
            pl.BlockSpec((1, _HP, _TILE, _DP), lambda b, t, s: (b, 0, t, 0)),
            pl.BlockSpec((1, _HP, _TILE, _DP), lambda b, t, s: (b, 0, t, 0)),
            pl.BlockSpec((1, _H, _TILE), lambda b, t, s: (b, 0, t)),
        ],
        scratch_shapes=[
            pltpu.VMEM((u, _DM), jnp.float32),
            pltpu.SemaphoreType.DMA,
        ],
    )
    return pl.pallas_call(
        _proj_m_kernel,
        grid_spec=grid_spec,
        out_shape=[
            jax.ShapeDtypeStruct((_B, _HP, _L, _DP), jnp.bfloat16),
            jax.ShapeDtypeStruct((_B, _HP, _L, _DP), jnp.bfloat16),
            jax.ShapeDtypeStruct((_B, _HP, _L, _DP), jnp.bfloat16),
            jax.ShapeDtypeStruct((_B, _H, _L), jnp.float32),
        ],
        compiler_params=pltpu.CompilerParams(
            dimension_semantics=("arbitrary", "arbitrary")),
    )(sample_idx, queries, keys, values, keys,
      wq, wk, wv, bq.reshape(1, _DM), bk.reshape(1, _DM), bv.reshape(1, _DM))


def _topk_kernel(m_ref, idx_ref, idxf_ref, m_scr, nsel):
    bh = m_ref.shape[0]
    m_scr[...] = m_ref[...]
    iota = jax.lax.broadcasted_iota(jnp.int32, (bh, _L), 1)
    col = jax.lax.broadcasted_iota(jnp.int32, (bh, nsel), 1)

    def body(i, c):
        m = m_scr[...]
        mx = jnp.max(m, axis=1, keepdims=True)
        idx = jnp.min(jnp.where(m == mx, iota, _L), axis=1)
        idx_ref[...] = jnp.where(col == i, idx[:, None], idx_ref[...])
        idxf_ref[...] = jnp.where(col == i, idx.astype(jnp.float32)[:, None],
                                  idxf_ref[...])
        m_scr[...] = jnp.where(iota == idx[:, None], -jnp.inf, m)
        return c
    jax.lax.fori_loop(0, nsel, body, 0)


def _topk(m2, nsel):
    bh = m2.shape[0]
    return pl.pallas_call(
        lambda m_ref, i_ref, f_ref, m_scr: _topk_kernel(
            m_ref, i_ref, f_ref, m_scr, nsel),
        grid=(1,),
        in_specs=[pl.BlockSpec((bh, _L), lambda i: (0, 0))],
        out_specs=[pl.BlockSpec((bh, nsel), lambda i: (0, 0)),
                   pl.BlockSpec((bh, nsel), lambda i: (0, 0))],
        out_shape=[jax.ShapeDtypeStruct((bh, nsel), jnp.int32),
                   jax.ShapeDtypeStruct((bh, nsel), jnp.float32)],
        scratch_shapes=[pltpu.VMEM((bh, _L), jnp.float32)],
    )(m2)


def _attn_kernel(q_ref, k_ref, v_ref, idxf_ref,
                 attn_ref, upd_ref, vmean_ref, nsel, scale):
    n2 = 2 * nsel

    idv = jnp.transpose(idxf_ref[0, 0])
    lane = jax.lax.broadcasted_iota(jnp.int32, (n2, _L), 1).astype(jnp.float32)
    selq = jnp.where(idv == lane, 1.0, 0.0).astype(jnp.bfloat16)
    qs_bd = _dot(selq, q_ref[0, 0], ((1,), (0,)))
    rows = jax.lax.broadcasted_iota(jnp.int32, (n2, _DP), 0)
    cols = jax.lax.broadcasted_iota(jnp.int32, (n2, _DP), 1)
    mask = (rows < nsel) == (cols < _DK)
    qs_bd = jnp.where(mask, qs_bd, 0.0).astype(jnp.bfloat16)

    scores = _dot(qs_bd, k_ref[0, 0], ((1,), (1,))) * scale
    mx = jnp.max(scores, axis=1, keepdims=True)
    p = jnp.exp(scores - mx)
    a = p / jnp.sum(p, axis=1, keepdims=True)
    attn_ref[0, 0] = a[:nsel]
    attn_ref[0, 1] = a[nsel:]
    upd_ref[0, 0] = _dot(a.astype(jnp.bfloat16), v_ref[0, 0],
                         ((1,), (0,)))
    vmean_ref[0, 0, 0] = jnp.mean(v_ref[0, 0].astype(jnp.float32), axis=0)


def _attn(idxf_bd, q_pair, k_pair, v_pair, nsel, scale):
    n2 = 2 * nsel
    kern = lambda *a: _attn_kernel(*a, nsel=nsel, scale=scale)
    return pl.pallas_call(
        kern,
        grid=(_B, _HP),
        in_specs=[
            pl.BlockSpec((1, 1, _L, _DP), lambda b, g: (b, g, 0, 0)),
            pl.BlockSpec((1, 1, _L, _DP), lambda b, g: (b, g, 0, 0)),
            pl.BlockSpec((1, 1, _L, _DP), lambda b, g: (b, g, 0, 0)),
            pl.BlockSpec((1, 1, 1, n2), lambda b, g: (b, g, 0, 0)),
        ],
        out_specs=[
            pl.BlockSpec((1, 2, nsel, _L), lambda b, g: (b, g, 0, 0)),
            pl.BlockSpec((1, 1, n2, _DP), lambda b, g: (b, g, 0, 0)),
            pl.BlockSpec((1, 1, 1, _DP), lambda b, g: (b, g, 0, 0)),
        ],
        out_shape=[
            jax.ShapeDtypeStruct((_B, _H, nsel, _L), jnp.float32),
            jax.ShapeDtypeStruct((_B, _HP, n2, _DP), jnp.float32),
            jax.ShapeDtypeStruct((_B, _HP, 1, _DP), jnp.float32),
        ],
        compiler_params=pltpu.CompilerParams(
            dimension_semantics=("arbitrary", "arbitrary")),
    )(q_pair, k_pair, v_pair, idxf_bd)


def _out_kernel(upd_ref, vm_ref, vmf_ref, wo_ref, bo_ref, idxf_ref,
                out_ref, delta_scr, base_scr, nsel, lh):
    t = pl.program_id(1)
    n2 = 2 * nsel

    @pl.when(t == 0)
    def _():
        base_scr[...] = _dot(vmf_ref[0], wo_ref[...], ((1,), (1,))) + bo_ref[...]
        rows = jax.lax.broadcasted_iota(jnp.int32, (n2, _DP), 0)
        cols = jax.lax.broadcasted_iota(jnp.int32, (n2, _DP), 1)
        mask = (rows < nsel) == (cols < _DK)
        for g in range(_HP):
            du = jnp.where(mask, upd_ref[0, g] - vm_ref[0, g], 0.0)
            wo_g = wo_ref[:, g * _DP:(g + 1) * _DP]
            delta_scr[g * n2:(g + 1) * n2, :] = _dot(du, wo_g, ((1,), (1,)))

    nrow = delta_scr.shape[0]
    lo = t * lh
    rowv = (jax.lax.broadcasted_iota(jnp.int32, (lh, nrow), 0)
            + lo).astype(jnp.float32)
    sel = jnp.where(idxf_ref[0, 0][None, :] == rowv, 1.0, 0.0)
    st = _dot(sel, delta_scr[...], ((1,), (0,)))
    out_ref[0] = st + base_scr[...]


def _out(idxf, upd_pair, vmean_pair, wo, bo, nsel):
    nt = 8
    lh = _L // nt
    n2 = 2 * nsel
    kern = lambda *a: _out_kernel(*a, nsel=nsel, lh=lh)
    return pl.pallas_call(
        kern,
        grid=(_B, nt),
        in_specs=[
            pl.BlockSpec((1, _HP, n2, _DP), lambda b, t: (b, 0, 0, 0)),
            pl.BlockSpec((1, _HP, 1, _DP), lambda b, t: (b, 0, 0, 0)),
            pl.BlockSpec((1, 1, _DM), lambda b, t: (b, 0, 0)),
            pl.BlockSpec((_DM, _DM), lambda b, t: (0, 0)),
            pl.BlockSpec((1, _DM), lambda b, t: (0, 0)),
            pl.BlockSpec((1, 1, _H * nsel), lambda b, t: (b, 0, 0)),
        ],
        out_specs=pl.BlockSpec((1, lh, _DM), lambda b, t: (b, t, 0)),
        out_shape=jax.ShapeDtypeStruct((_B, _L, _DM), jnp.float32),
        scratch_shapes=[
            pltpu.VMEM((_H * nsel, _DM), jnp.float32),
            pltpu.VMEM((1, _DM), jnp.float32),
        ],
        compiler_params=pltpu.CompilerParams(
            dimension_semantics=("arbitrary", "arbitrary")),
    )(upd_pair, vmean_pair, vmean_pair.reshape(_B, 1, _DM),
      wo, bo.reshape(1, _DM), idxf)


def kernel(queries, keys, values, Wq, bq, Wk, bk, Wv, bv, Wo, bo):
    bsz, l_q, _ = queries.shape
    _, l_k, _ = keys.shape
    u = min(_FACTOR * int(np.ceil(np.log(l_k + 1))), l_k)
    nsel = min(_FACTOR * int(np.ceil(np.log(l_q + 1))), l_q)
    scale = 1.0 / math.sqrt(_DK)
    sample_idx = jax.random.randint(jax.random.key(42), (u,), 0, l_k)

    k_pair, v_pair, q_pair, m = _proj_m(sample_idx, queries, keys, values,
                                        Wq, Wk, Wv, bq, bk, bv, u)
    m_top2, m_topf2 = _topk(m.reshape(_B * _H, _L), nsel)
    idxf = m_topf2.reshape(_B, 1, _H * nsel)
    idxf_bd = m_topf2.reshape(_B, _HP, 1, 2 * nsel)
    attn, upd_pair, vmean_pair = _attn(idxf_bd, q_pair, k_pair, v_pair,
                                       nsel, scale)
    output = _out(idxf, upd_pair, vmean_pair, Wo, bo, nsel)
    import os as _os
    _stage = 4
    if _stage == 1:
        return m
    if _stage == 2:
        return m_top2
    if _stage == 3:
        return (attn, upd_pair)
    return (output, attn)


# speedup vs baseline: 1.1192x; 1.0026x over previous
import math

import jax
import jax.numpy as jnp
import numpy as np
from jax.experimental import pallas as pl
from jax.experimental.pallas import tpu as pltpu

_B = 4
_L = 8192
_DM = 1024
_H = 16
_DK = _DM // _H
_HP = _H // 2
_DP = 2 * _DK
_FACTOR = 5
_TILE = 512
_PREC = jax.lax.Precision.DEFAULT


def _dot(a, b, dims, prec=_PREC):
    return jax.lax.dot_general(
        a, b, (dims, ((), ())), precision=prec,
        preferred_element_type=jnp.float32)


_MPREC = jax.lax.Precision.DEFAULT


def _proj_m_kernel(idx_ref, q_ref, k_ref, v_ref, khbm_ref,
                   wq_ref, wk_ref, wv_ref, bq_ref, bk_ref, bv_ref,
                   kout_ref, vout_ref, qout_ref, m_ref, ks_scr, sem):
    b = pl.program_id(0)
    t = pl.program_id(1)
    u = ks_scr.shape[0]
    tile = q_ref.shape[1]

    @pl.when(t == 0)
    def _():
        def start(i, c):
            idx = idx_ref[i]
            pltpu.make_async_copy(khbm_ref.at[b, pl.ds(idx, 1), :],
                                  ks_scr.at[pl.ds(i, 1), :], sem).start()
            return c
        jax.lax.fori_loop(0, u, start, 0)

        def wait(i, c):
            pltpu.make_async_copy(khbm_ref.at[b, pl.ds(0, 1), :],
                                  ks_scr.at[pl.ds(i, 1), :], sem).wait()
            return c
        jax.lax.fori_loop(0, u, wait, 0)
        ks_scr[...] = _dot(ks_scr[...], wk_ref[...], ((1,), (1,)), _MPREC) + bk_ref[...]

    kt = (_dot(k_ref[0], wk_ref[...], ((1,), (1,))) + bk_ref[...]
          ).astype(jnp.bfloat16)
    vt = (_dot(v_ref[0], wv_ref[...], ((1,), (1,))) + bv_ref[...]
          ).astype(jnp.bfloat16)
    for g in range(_HP):
        kout_ref[0, g] = kt[:, g * _DP:(g + 1) * _DP]
        vout_ref[0, g] = vt[:, g * _DP:(g + 1) * _DP]

    qt = _dot(q_ref[0], wq_ref[...], ((1,), (1,)), _MPREC) + bq_ref[...]
    qt_bf = qt.astype(jnp.bfloat16)
    for g in range(_HP):
        qout_ref[0, g] = qt_bf[:, g * _DP:(g + 1) * _DP]
    ks = ks_scr[...]
    u2 = 2 * u
    colu = jax.lax.broadcasted_iota(jnp.int32, (u, _DP), 1)
    rows = jax.lax.broadcasted_iota(jnp.int32, (u2, tile), 0)
    lo = rows < u
    for g in range(_HP):
        ks_pair = ks[:, g * _DP:(g + 1) * _DP]
        top = jnp.where(colu < _DK, ks_pair, 0.0)
        bot = jnp.where(colu >= _DK, ks_pair, 0.0)
        ks_bd = jnp.concatenate([top, bot], axis=0)
        st = _dot(ks_bd, qt[:, g * _DP:(g + 1) * _DP], ((1,), (1,)), _MPREC)
        m_ref[0, 2 * g, :] = (jnp.max(jnp.where(lo, st, -jnp.inf), axis=0)
                              - jnp.sum(jnp.where(lo, st, 0.0), axis=0) / u)
        m_ref[0, 2 * g + 1, :] = (jnp.max(jnp.where(lo, -jnp.inf, st), axis=0)
                                  - jnp.sum(jnp.where(lo, 0.0, st), axis=0) / u)


def _proj_m(sample_idx, queries, keys, values, wq, wk, wv, bq, bk, bv, u):
    nt = _L // _TILE
    grid_spec = pltpu.PrefetchScalarGridSpec(
        num_scalar_prefetch=1,
        grid=(_B, nt),
        in_specs=[
            pl.BlockSpec((1, _TILE, _DM), lambda b, t, s: (b, t, 0)),
            pl.BlockSpec((1, _TILE, _DM), lambda b, t, s: (b, t, 0)),
            pl.BlockSpec((1, _TILE, _DM), lambda b, t, s: (b, t, 0)),
            pl.BlockSpec(memory_space=pl.ANY),
            pl.BlockSpec((_DM, _DM), lambda b, t, s: (0, 0)),
            pl.BlockSpec((_DM, _DM), lambda b, t, s: (0, 0)),
            pl.BlockSpec((_DM, _DM), lambda b, t, s: (0, 0)),
            pl.BlockSpec((1, _DM), lambda b, t, s: (0, 0)),
            pl.BlockSpec((1, _DM), lambda b, t, s: (0, 0)),
            pl.BlockSpec((1, _DM), lambda b, t, s: (0, 0)),
        ],
        out_specs=[
            pl.BlockSpec((1, _HP, _TILE, _DP), lambda b, t, s: (b, 0, t, 0)),
            pl.BlockSpec((1, _HP, _TILE, _DP), lambda b, t, s: (b, 0, t, 0)),
            pl.BlockSpec((1, _HP, _TILE, _DP), lambda b, t, s: (b, 0, t, 0)),
            pl.BlockSpec((1, _H, _TILE), lambda b, t, s: (b, 0, t)),
        ],
        scratch_shapes=[
            pltpu.VMEM((u, _DM), jnp.float32),
            pltpu.SemaphoreType.DMA,
        ],
    )
    return pl.pallas_call(
        _proj_m_kernel,
        grid_spec=grid_spec,
        out_shape=[
            jax.ShapeDtypeStruct((_B, _HP, _L, _DP), jnp.bfloat16),
            jax.ShapeDtypeStruct((_B, _HP, _L, _DP), jnp.bfloat16),
            jax.ShapeDtypeStruct((_B, _HP, _L, _DP), jnp.bfloat16),
            jax.ShapeDtypeStruct((_B, _H, _L), jnp.float32),
        ],
        compiler_params=pltpu.CompilerParams(
            dimension_semantics=("arbitrary", "arbitrary")),
    )(sample_idx, queries, keys, values, keys,
      wq, wk, wv, bq.reshape(1, _DM), bk.reshape(1, _DM), bv.reshape(1, _DM))


def _topk_kernel(m_ref, idx_ref, idxf_ref, m_scr, nsel):
    bh = m_ref.shape[0]
    m_scr[...] = m_ref[...]
    iota = jax.lax.broadcasted_iota(jnp.int32, (bh, _L), 1)
    col = jax.lax.broadcasted_iota(jnp.int32, (bh, nsel), 1)

    def body(i, c):
        m = m_scr[...]
        mx = jnp.max(m, axis=1, keepdims=True)
        idx = jnp.min(jnp.where(m == mx, iota, _L), axis=1)
        idx_ref[...] = jnp.where(col == i, idx[:, None], idx_ref[...])
        idxf_ref[...] = jnp.where(col == i, idx.astype(jnp.float32)[:, None],
                                  idxf_ref[...])
        m_scr[...] = jnp.where(iota == idx[:, None], -jnp.inf, m)
        return c
    jax.lax.fori_loop(0, nsel, body, 0)


def _topk(m2, nsel):
    bh = m2.shape[0]
    return pl.pallas_call(
        lambda m_ref, i_ref, f_ref, m_scr: _topk_kernel(
            m_ref, i_ref, f_ref, m_scr, nsel),
        grid=(1,),
        in_specs=[pl.BlockSpec((bh, _L), lambda i: (0, 0))],
        out_specs=[pl.BlockSpec((bh, nsel), lambda i: (0, 0)),
                   pl.BlockSpec((bh, nsel), lambda i: (0, 0))],
        out_shape=[jax.ShapeDtypeStruct((bh, nsel), jnp.int32),
                   jax.ShapeDtypeStruct((bh, nsel), jnp.float32)],
        scratch_shapes=[pltpu.VMEM((bh, _L), jnp.float32)],
    )(m2)


def _attn_kernel(q_ref, k_ref, v_ref, idxf_ref,
                 attn_ref, upd_ref, vmean_ref, nsel, scale):
    n2 = 2 * nsel

    idv = jnp.transpose(idxf_ref[0, 0])
    lane = jax.lax.broadcasted_iota(jnp.int32, (n2, _L), 1).astype(jnp.float32)
    selq = jnp.where(idv == lane, 1.0, 0.0).astype(jnp.bfloat16)
    qs_bd = _dot(selq, q_ref[0, 0], ((1,), (0,)))
    rows = jax.lax.broadcasted_iota(jnp.int32, (n2, _DP), 0)
    cols = jax.lax.broadcasted_iota(jnp.int32, (n2, _DP), 1)
    mask = (rows < nsel) == (cols < _DK)
    qs_bd = jnp.where(mask, qs_bd, 0.0).astype(jnp.bfloat16)

    scores = _dot(qs_bd, k_ref[0, 0], ((1,), (1,))) * scale
    mx = jnp.max(scores, axis=1, keepdims=True)
    p = jnp.exp(scores - mx)
    a = p * (1.0 / jnp.sum(p, axis=1, keepdims=True))
    attn_ref[0, 0] = a[:nsel]
    attn_ref[0, 1] = a[nsel:]
    upd_ref[0, 0] = _dot(a.astype(jnp.bfloat16), v_ref[0, 0],
                         ((1,), (0,)))
    vmean_ref[0, 0, 0] = jnp.mean(v_ref[0, 0].astype(jnp.float32), axis=0)


def _attn(idxf_bd, q_pair, k_pair, v_pair, nsel, scale):
    n2 = 2 * nsel
    kern = lambda *a: _attn_kernel(*a, nsel=nsel, scale=scale)
    return pl.pallas_call(
        kern,
        grid=(_B, _HP),
        in_specs=[
            pl.BlockSpec((1, 1, _L, _DP), lambda b, g: (b, g, 0, 0)),
            pl.BlockSpec((1, 1, _L, _DP), lambda b, g: (b, g, 0, 0)),
            pl.BlockSpec((1, 1, _L, _DP), lambda b, g: (b, g, 0, 0)),
            pl.BlockSpec((1, 1, 1, n2), lambda b, g: (b, g, 0, 0)),
        ],
        out_specs=[
            pl.BlockSpec((1, 2, nsel, _L), lambda b, g: (b, g, 0, 0)),
            pl.BlockSpec((1, 1, n2, _DP), lambda b, g: (b, g, 0, 0)),
            pl.BlockSpec((1, 1, 1, _DP), lambda b, g: (b, g, 0, 0)),
        ],
        out_shape=[
            jax.ShapeDtypeStruct((_B, _H, nsel, _L), jnp.float32),
            jax.ShapeDtypeStruct((_B, _HP, n2, _DP), jnp.float32),
            jax.ShapeDtypeStruct((_B, _HP, 1, _DP), jnp.float32),
        ],
        compiler_params=pltpu.CompilerParams(
            dimension_semantics=("arbitrary", "arbitrary")),
    )(q_pair, k_pair, v_pair, idxf_bd)


def _out_kernel(upd_ref, vm_ref, vmf_ref, wo_ref, bo_ref, idxf_ref,
                out_ref, delta_scr, base_scr, nsel, lh):
    t = pl.program_id(1)
    n2 = 2 * nsel

    @pl.when(t == 0)
    def _():
        base_scr[...] = _dot(vmf_ref[0], wo_ref[...], ((1,), (1,))) + bo_ref[...]
        rows = jax.lax.broadcasted_iota(jnp.int32, (n2, _DP), 0)
        cols = jax.lax.broadcasted_iota(jnp.int32, (n2, _DP), 1)
        mask = (rows < nsel) == (cols < _DK)
        for g in range(_HP):
            du = jnp.where(mask, upd_ref[0, g] - vm_ref[0, g], 0.0)
            wo_g = wo_ref[:, g * _DP:(g + 1) * _DP]
            delta_scr[g * n2:(g + 1) * n2, :] = _dot(du, wo_g, ((1,), (1,)))

    nrow = delta_scr.shape[0]
    lo = t * lh
    rowv = (jax.lax.broadcasted_iota(jnp.int32, (lh, nrow), 0)
            + lo).astype(jnp.float32)
    sel = jnp.where(idxf_ref[0, 0][None, :] == rowv, 1.0, 0.0)
    st = _dot(sel, delta_scr[...], ((1,), (0,)))
    out_ref[0] = st + base_scr[...]


def _out(idxf, upd_pair, vmean_pair, wo, bo, nsel):
    nt = 8
    lh = _L // nt
    n2 = 2 * nsel
    kern = lambda *a: _out_kernel(*a, nsel=nsel, lh=lh)
    return pl.pallas_call(
        kern,
        grid=(_B, nt),
        in_specs=[
            pl.BlockSpec((1, _HP, n2, _DP), lambda b, t: (b, 0, 0, 0)),
            pl.BlockSpec((1, _HP, 1, _DP), lambda b, t: (b, 0, 0, 0)),
            pl.BlockSpec((1, 1, _DM), lambda b, t: (b, 0, 0)),
            pl.BlockSpec((_DM, _DM), lambda b, t: (0, 0)),
            pl.BlockSpec((1, _DM), lambda b, t: (0, 0)),
            pl.BlockSpec((1, 1, _H * nsel), lambda b, t: (b, 0, 0)),
        ],
        out_specs=pl.BlockSpec((1, lh, _DM), lambda b, t: (b, t, 0)),
        out_shape=jax.ShapeDtypeStruct((_B, _L, _DM), jnp.float32),
        scratch_shapes=[
            pltpu.VMEM((_H * nsel, _DM), jnp.float32),
            pltpu.VMEM((1, _DM), jnp.float32),
        ],
        compiler_params=pltpu.CompilerParams(
            dimension_semantics=("arbitrary", "arbitrary")),
    )(upd_pair, vmean_pair, vmean_pair.reshape(_B, 1, _DM),
      wo, bo.reshape(1, _DM), idxf)


def kernel(queries, keys, values, Wq, bq, Wk, bk, Wv, bv, Wo, bo):
    bsz, l_q, _ = queries.shape
    _, l_k, _ = keys.shape
    u = min(_FACTOR * int(np.ceil(np.log(l_k + 1))), l_k)
    nsel = min(_FACTOR * int(np.ceil(np.log(l_q + 1))), l_q)
    scale = 1.0 / math.sqrt(_DK)
    sample_idx = jax.random.randint(jax.random.key(42), (u,), 0, l_k)

    k_pair, v_pair, q_pair, m = _proj_m(sample_idx, queries, keys, values,
                                        Wq, Wk, Wv, bq, bk, bv, u)
    m_top2, m_topf2 = _topk(m.reshape(_B * _H, _L), nsel)
    idxf = m_topf2.reshape(_B, 1, _H * nsel)
    idxf_bd = m_topf2.reshape(_B, _HP, 1, 2 * nsel)
    attn, upd_pair, vmean_pair = _attn(idxf_bd, q_pair, k_pair, v_pair,
                                       nsel, scale)
    output = _out(idxf, upd_pair, vmean_pair, Wo, bo, nsel)
    import os as _os
    _stage = 4
    if _stage == 1:
        return m
    if _stage == 2:
        return m_top2
    if _stage == 3:
        return (attn, upd_pair)
    return (output, attn)
